# baseline (speedup 1.0000x reference)
import functools

import jax
import jax.numpy as jnp
from jax import lax
from jax.experimental import pallas as pl
from jax.experimental.pallas import tpu as pltpu
from jax.experimental.pallas import tpu_sc as plsc

_NC = 2
_NS = 16
_NW = _NC * _NS
_L = 16

_B = 16384
_BPW = _B // _NW
_NV = _BPW // _L

_UROWS = 1500
_VROWS = 2000
_D = 3
_TAB = (_UROWS + _VROWS) * _D


@functools.partial(
    pl.kernel,
    out_type=jax.ShapeDtypeStruct((_B,), jnp.float32),
    mesh=plsc.VectorSubcoreMesh(core_axis_name="c", subcore_axis_name="s"),
    compiler_params=pltpu.CompilerParams(
        needs_layout_passes=False, use_tc_tiling_on_sc=False),
    scratch_types=[
        pltpu.VMEM((_TAB,), jnp.float32),
        pltpu.VMEM((_BPW,), jnp.int32),
        pltpu.VMEM((_BPW,), jnp.int32),
        pltpu.VMEM((_BPW,), jnp.float32),
        pltpu.SemaphoreType.DMA,
    ],
)
def _mf_kernel(idx_hbm, tab_hbm, out_hbm, tab_v, ui_v, vi_v, out_v, sem):
    wid = lax.axis_index("s") * _NC + lax.axis_index("c")
    base = wid * _BPW

    ct = pltpu.make_async_copy(tab_hbm, tab_v, sem)
    ci = pltpu.make_async_copy(idx_hbm.at[pl.ds(base, _BPW)], ui_v, sem)
    cj = pltpu.make_async_copy(idx_hbm.at[pl.ds(_B + base, _BPW)], vi_v, sem)
    ct.start()
    ci.start()
    cj.start()
    ct.wait()
    ci.wait()
    cj.wait()

    @plsc.parallel_loop(0, _NV)
    def _(i):
        off = pl.multiple_of(i * _L, _L)
        ub = ui_v[pl.ds(off, _L)] * _D
        vb = vi_v[pl.ds(off, _L)] * _D + (_UROWS * _D)
        acc = plsc.load_gather(tab_v, [ub]) * plsc.load_gather(tab_v, [vb])
        for d in range(1, _D):
            acc = acc + (plsc.load_gather(tab_v, [ub + d])
                         * plsc.load_gather(tab_v, [vb + d]))
        out_v[pl.ds(off, _L)] = acc

    pltpu.sync_copy(out_v, out_hbm.at[pl.ds(base, _BPW)])


def kernel(data, user_factors, item_factors):
    idx = data.astype(jnp.int32).reshape(-1)
    tab = jnp.concatenate([user_factors.reshape(-1),
                           item_factors.reshape(-1)])
    return _mf_kernel(idx, tab)


# speedup vs baseline: 1.0436x; 1.0436x over previous
import functools

import jax
import jax.numpy as jnp
from jax import lax
from jax.experimental import pallas as pl
from jax.experimental.pallas import tpu as pltpu
from jax.experimental.pallas import tpu_sc as plsc

_NC = 2
_NS = 16
_NW = _NC * _NS
_L = 16

_B = 16384
_BPW = _B // _NW
_NV = _BPW // _L

_UROWS = 1500
_VROWS = 2000
_D = 3
_TAB = (_UROWS + _VROWS) * _D
_TABPAD = _TAB + 4


@functools.partial(
    pl.kernel,
    out_type=jax.ShapeDtypeStruct((_B,), jnp.float32),
    mesh=plsc.VectorSubcoreMesh(core_axis_name="c", subcore_axis_name="s"),
    compiler_params=pltpu.CompilerParams(
        needs_layout_passes=False, use_tc_tiling_on_sc=False),
    scratch_types=[
        pltpu.VMEM((_TAB,), jnp.int32),
        pltpu.VMEM((_BPW,), jnp.int32),
        pltpu.VMEM((_BPW,), jnp.int32),
        pltpu.VMEM((_BPW,), jnp.float32),
        pltpu.SemaphoreType.DMA,
    ],
)
def _mf_kernel(buf_hbm, out_hbm, tab_v, ui_v, vi_v, out_v, sem):
    wid = lax.axis_index("s") * _NC + lax.axis_index("c")
    base = wid * _BPW

    ct = pltpu.make_async_copy(buf_hbm.at[pl.ds(0, _TAB)], tab_v, sem)
    ci = pltpu.make_async_copy(
        buf_hbm.at[pl.ds(_TABPAD + base, _BPW)], ui_v, sem)
    cj = pltpu.make_async_copy(
        buf_hbm.at[pl.ds(_TABPAD + _B + base, _BPW)], vi_v, sem)
    ct.start()
    ci.start()
    cj.start()
    ct.wait()
    ci.wait()
    cj.wait()

    @plsc.parallel_loop(0, _NV)
    def _(i):
        off = pl.multiple_of(i * _L, _L)
        ub = ui_v[pl.ds(off, _L)] * _D
        vb = vi_v[pl.ds(off, _L)] * _D + (_UROWS * _D)
        acc = None
        for d in range(_D):
            u = plsc.bitcast(plsc.load_gather(tab_v, [ub + d]), jnp.float32)
            v = plsc.bitcast(plsc.load_gather(tab_v, [vb + d]), jnp.float32)
            acc = u * v if acc is None else acc + u * v
        out_v[pl.ds(off, _L)] = acc

    pltpu.sync_copy(out_v, out_hbm.at[pl.ds(base, _BPW)])


def kernel(data, user_factors, item_factors):
    buf = jnp.concatenate([
        jax.lax.bitcast_convert_type(user_factors, jnp.int32).reshape(-1),
        jax.lax.bitcast_convert_type(item_factors, jnp.int32).reshape(-1),
        jnp.zeros((_TABPAD - _TAB,), jnp.int32),
        data.astype(jnp.int32).reshape(-1),
    ])
    return _mf_kernel(buf)
